# Initial kernel scaffold; baseline (speedup 1.0000x reference)
import jax
import jax.numpy as jnp
from jax import lax
from jax.experimental import pallas as pl
from jax.experimental.pallas import tpu as pltpu
from jax.experimental.pallas import tpu_sc as plsc

N = 10000
E = 320000
IN_FEATS = 128
HIDDEN = 128
N_CLASSES = 40
ALPHA = 0.1
K = 10

NC = 2
NS = 16
NW = NC * NS
CH = 128
NCHUNK = 79
EPT = CH * NCHUNK
EP = EPT * NW
NP = 10240
RPT = NP // NW
HALF = N_CLASSES // 2
NPH = NP * HALF
FPT = RPT * HALF
LANES = 16
MAGIC = 52429


def _mesh():
    return plsc.VectorSubcoreMesh(core_axis_name="c", subcore_axis_name="s")


def _count_body(src_hbm, dst_hbm, out_hbm, sidx_v, didx_v, cnt_v):
    cid = lax.axis_index("c")
    sid = lax.axis_index("s")
    wid = cid * NS + sid

    pltpu.sync_copy(src_hbm.at[wid], sidx_v)
    pltpu.sync_copy(dst_hbm.at[wid], didx_v)

    zeros = jnp.zeros((LANES,), jnp.float32)
    ones = jnp.ones((LANES,), jnp.float32)

    def zero_row(i, _):
        cnt_v[0, pl.ds(i * LANES, LANES)] = zeros
        cnt_v[1, pl.ds(i * LANES, LANES)] = zeros
        return 0

    lax.fori_loop(0, NP // LANES, zero_row, 0)

    def count_chunk(j, _):
        def one_vec(k, _):
            s = sidx_v[j, pl.ds(k * LANES, LANES)]
            d = didx_v[j, pl.ds(k * LANES, LANES)]
            plsc.addupdate_scatter(cnt_v.at[0], [s], ones)
            plsc.addupdate_scatter(cnt_v.at[1], [d], ones)
            return 0
        return lax.fori_loop(0, CH // LANES, one_vec, 0)

    lax.fori_loop(0, NCHUNK, count_chunk, 0)
    pltpu.sync_copy(cnt_v, out_hbm.at[wid])


def _count_call(src3, dst3):
    return pl.kernel(
        _count_body,
        out_type=jax.ShapeDtypeStruct((NW, 2, NP), jnp.float32),
        mesh=_mesh(),
        scratch_types=[
            pltpu.VMEM((NCHUNK, CH), jnp.int32),
            pltpu.VMEM((NCHUNK, CH), jnp.int32),
            pltpu.VMEM((2, NP), jnp.float32),
        ],
    )(src3, dst3)


def _mlp_body(x_ref, w1_ref, b1_ref, w2_ref, b2_ref, cnt_ref,
              h0_ref, g0_ref, sc_ref, fs_ref):
    x = x_ref[...]
    h = jnp.maximum(jnp.dot(x, w1_ref[...],
                            preferred_element_type=jnp.float32)
                    + b1_ref[...], 0.0)
    h0 = jnp.dot(h, w2_ref[...], preferred_element_type=jnp.float32) \
        + b2_ref[...]
    tot = jnp.sum(cnt_ref[...], axis=2)
    tot = jnp.maximum(tot, 1.0)
    out_norm = lax.rsqrt(tot[:, 0:1])
    in_norm = lax.rsqrt(tot[:, 1:2])
    h0_ref[...] = h0
    g0_ref[...] = h0 * out_norm
    sc_ref[...] = jnp.broadcast_to((1.0 - ALPHA) * out_norm * in_norm,
                                   h0.shape)
    fs_ref[...] = jnp.broadcast_to((1.0 - ALPHA) * in_norm, h0.shape)


def _mlp_call(xp, W1, b1, W2, b2, cntT):
    BLK = 512
    grid = NP // BLK
    ospec = pl.BlockSpec((BLK, N_CLASSES), lambda i: (i, 0))
    oshape = jax.ShapeDtypeStruct((NP, N_CLASSES), jnp.float32)
    return pl.pallas_call(
        _mlp_body,
        grid=(grid,),
        in_specs=[
            pl.BlockSpec((BLK, IN_FEATS), lambda i: (i, 0)),
            pl.BlockSpec((IN_FEATS, HIDDEN), lambda i: (0, 0)),
            pl.BlockSpec((1, HIDDEN), lambda i: (0, 0)),
            pl.BlockSpec((HIDDEN, N_CLASSES), lambda i: (0, 0)),
            pl.BlockSpec((1, N_CLASSES), lambda i: (0, 0)),
            pl.BlockSpec((BLK, 2, NW), lambda i: (i, 0, 0)),
        ],
        out_specs=[ospec, ospec, ospec, ospec],
        out_shape=[oshape, oshape, oshape, oshape],
    )(xp, W1, b1, W2, b2, cntT)


def _prop_body(g_hbm, sc_hbm, b_hbm, src_hbm, dst_hbm, zz_hbm, out_hbm,
               sidx_v, didx_v, row_v, agg_v, scf_v, bf_v, outf_v, agg_sp,
               sem):
    cid = lax.axis_index("c")
    sid = lax.axis_index("s")
    wid = cid * NS + sid
    r0 = sid * RPT

    pltpu.sync_copy(src_hbm.at[wid], sidx_v)
    pltpu.sync_copy(dst_hbm.at[wid], didx_v)

    pltpu.sync_copy(zz_hbm, agg_sp.at[pl.ds(r0, RPT)])
    plsc.subcore_barrier()

    gh = g_hbm.at[cid]

    def chunk(j, _):
        pltpu.async_copy(gh.at[sidx_v.at[j]], row_v, sem).wait()
        pltpu.sync_copy(row_v, agg_sp.at[didx_v.at[j]], add=True)
        return 0

    lax.fori_loop(0, NCHUNK, chunk, 0)
    plsc.subcore_barrier()

    pltpu.sync_copy(agg_sp.at[pl.ds(r0, RPT)], agg_v)
    pltpu.sync_copy(sc_hbm.at[cid, pl.ds(sid * FPT, FPT)], scf_v)
    pltpu.sync_copy(b_hbm.at[cid, pl.ds(sid * FPT, FPT)], bf_v)

    lane = lax.iota(jnp.int32, LANES)

    def upd_vec(i, _):
        sl = pl.ds(i * LANES, LANES)
        f = i * LANES + lane
        row = lax.shift_right_logical(f * MAGIC, 20)
        col = f - row * 20
        a = plsc.load_gather(agg_v, [row, col])
        outf_v[sl] = scf_v[sl] * a + bf_v[sl]
        return 0

    lax.fori_loop(0, FPT // LANES, upd_vec, 0)
    pltpu.sync_copy(outf_v, out_hbm.at[cid, pl.ds(sid * FPT, FPT)])


def _prop_call(g, scale_f, bias_f, src3, dst3, zz):
    return pl.kernel(
        _prop_body,
        out_type=jax.ShapeDtypeStruct((NC, NPH), jnp.float32),
        mesh=_mesh(),
        scratch_types=[
            pltpu.VMEM((NCHUNK, CH), jnp.int32),
            pltpu.VMEM((NCHUNK, CH), jnp.int32),
            pltpu.VMEM((CH, HALF), jnp.float32),
            pltpu.VMEM((RPT, HALF), jnp.float32),
            pltpu.VMEM((FPT,), jnp.float32),
            pltpu.VMEM((FPT,), jnp.float32),
            pltpu.VMEM((FPT,), jnp.float32),
            pltpu.VMEM_SHARED((NP, HALF), jnp.float32),
            pltpu.SemaphoreType.DMA,
        ],
    )(g, scale_f, bias_f, src3, dst3, zz)


def _halves(a):
    return jnp.stack([a[:, :HALF], a[:, HALF:]], axis=0)


def kernel(features, edge_index, W1, b1, W2, b2):
    src = edge_index[0].astype(jnp.int32)
    dst = edge_index[1].astype(jnp.int32)
    pad = EP - E
    fill = jnp.full((pad,), NP - 1, jnp.int32)
    src3 = jnp.concatenate([src, fill]).reshape(NW, NCHUNK, CH)
    dst3 = jnp.concatenate([dst, fill]).reshape(NW, NCHUNK, CH)
    xp = jnp.pad(features.astype(jnp.float32), ((0, NP - N), (0, 0)))
    zz = jnp.zeros((RPT, HALF), jnp.float32)

    cnt = _count_call(src3, dst3)
    cntT = jnp.transpose(cnt, (2, 1, 0))
    h0, g0, scarr, fsarr = _mlp_call(
        xp, W1.astype(jnp.float32), b1.astype(jnp.float32).reshape(1, -1),
        W2.astype(jnp.float32), b2.astype(jnp.float32).reshape(1, -1), cntT)

    g = _halves(g0)
    sch = _halves(scarr).reshape(NC, NPH)
    biash = (ALPHA * _halves(g0)).reshape(NC, NPH)
    fsh = _halves(fsarr).reshape(NC, NPH)
    fbh = (ALPHA * _halves(h0)).reshape(NC, NPH)

    for _ in range(K - 1):
        g = _prop_call(g, sch, biash, src3, dst3, zz).reshape(NC, NP, HALF)
    out = _prop_call(g, fsh, fbh, src3, dst3, zz).reshape(NC, NP, HALF)
    res = jnp.concatenate([out[0], out[1]], axis=1)[:N]
    return res.astype(features.dtype)


# speedup vs baseline: 218.5956x; 218.5956x over previous
import jax
import jax.numpy as jnp
from jax import lax
from jax.experimental import pallas as pl
from jax.experimental.pallas import tpu as pltpu
from jax.experimental.pallas import tpu_sc as plsc
from jax._src.config import enable_x64 as _enable_x64

N = 10000
E = 320000
IN_FEATS = 128
HIDDEN = 128
N_CLASSES = 40
ALPHA = 0.1
K = 10

NC = 2
NS = 16
NW = NC * NS
CH = 128
NCHUNK = 158
EPT = CH * NCHUNK
EP = EPT * NS
NP = 10240
RPT = NP // NS
HALF = N_CLASSES // 2
HP = 32
LANES = 16
I32 = jnp.int32


def _mesh():
    return plsc.VectorSubcoreMesh(core_axis_name="c", subcore_axis_name="s")


def _count_body(src_hbm, dst_hbm, out_hbm, sidx_v, didx_v, cs_v, cd_v):
    cid = lax.axis_index("c")
    sid = lax.axis_index("s")
    wid = cid * I32(NS) + sid

    pltpu.sync_copy(src_hbm.at[sid], sidx_v)
    pltpu.sync_copy(dst_hbm.at[sid], didx_v)

    zeros = jnp.zeros((LANES,), jnp.float32)
    ones = jnp.ones((LANES,), jnp.float32)

    def zero_row(i, _):
        cs_v[pl.ds(i * I32(LANES), LANES)] = zeros
        cd_v[pl.ds(i * I32(LANES), LANES)] = zeros
        return 0

    lax.fori_loop(0, NP // LANES, zero_row, 0)

    def count_chunk(j, _):
        def one_vec(k, _):
            s = sidx_v[j, pl.ds(k * I32(LANES), LANES)]
            d = didx_v[j, pl.ds(k * I32(LANES), LANES)]
            plsc.addupdate_scatter(cs_v, [s], ones)
            plsc.addupdate_scatter(cd_v, [d], ones)
            return 0
        return lax.fori_loop(0, CH // LANES, one_vec, 0)

    lax.fori_loop(0, NCHUNK, count_chunk, 0)
    pltpu.sync_copy(cs_v, out_hbm.at[wid, 0])
    pltpu.sync_copy(cd_v, out_hbm.at[wid, 1])


def _count_call(src3, dst3):
    return pl.kernel(
        _count_body,
        out_type=jax.ShapeDtypeStruct((NW, 2, NP), jnp.float32),
        mesh=_mesh(),
        scratch_types=[
            pltpu.VMEM((NCHUNK, CH), jnp.int32),
            pltpu.VMEM((NCHUNK, CH), jnp.int32),
            pltpu.VMEM((NP,), jnp.float32),
            pltpu.VMEM((NP,), jnp.float32),
        ],
        compiler_params=pltpu.CompilerParams(needs_layout_passes=False),
    )(src3, dst3)


def _mlp_body(x_ref, w1_ref, b1_ref, w2_ref, b2_ref, cnt_ref,
              h0_ref, g0_ref, sc_ref, fs_ref):
    x = x_ref[...]
    h = jnp.maximum(jnp.dot(x, w1_ref[...],
                            preferred_element_type=jnp.float32)
                    + b1_ref[...], 0.0)
    h0 = jnp.dot(h, w2_ref[...], preferred_element_type=jnp.float32) \
        + b2_ref[...]
    tot = 0.5 * jnp.sum(cnt_ref[...], axis=2)
    tot = jnp.maximum(tot, 1.0)
    out_norm = lax.rsqrt(tot[:, 0:1])
    in_norm = lax.rsqrt(tot[:, 1:2])
    h0_ref[...] = h0
    g0_ref[...] = h0 * out_norm
    sc_ref[...] = jnp.broadcast_to((1.0 - ALPHA) * out_norm * in_norm,
                                   h0.shape)
    fs_ref[...] = jnp.broadcast_to((1.0 - ALPHA) * in_norm, h0.shape)


def _mlp_call(xp, W1, b1, W2, b2, cntT):
    BLK = 512
    grid = NP // BLK
    ospec = pl.BlockSpec((BLK, N_CLASSES), lambda i: (i, 0))
    oshape = jax.ShapeDtypeStruct((NP, N_CLASSES), jnp.float32)
    return pl.pallas_call(
        _mlp_body,
        grid=(grid,),
        in_specs=[
            pl.BlockSpec((BLK, IN_FEATS), lambda i: (i, 0)),
            pl.BlockSpec((IN_FEATS, HIDDEN), lambda i: (0, 0)),
            pl.BlockSpec((1, HIDDEN), lambda i: (0, 0)),
            pl.BlockSpec((HIDDEN, N_CLASSES), lambda i: (0, 0)),
            pl.BlockSpec((1, N_CLASSES), lambda i: (0, 0)),
            pl.BlockSpec((BLK, 2, NW), lambda i: (i, 0, 0)),
        ],
        out_specs=[ospec, ospec, ospec, ospec],
        out_shape=[oshape, oshape, oshape, oshape],
    )(xp, W1, b1, W2, b2, cntT)


def _prop_body(g_hbm, sc_hbm, b_hbm, src_hbm, dst_hbm, zz_hbm, out_hbm,
               sidx_v, didx_v, row_v, agg_v, scf_v, bf_v, agg_sp, sem):
    cid = lax.axis_index("c")
    sid = lax.axis_index("s")
    wid = cid * I32(NS) + sid
    r0 = sid * I32(RPT)

    pltpu.sync_copy(src_hbm.at[sid], sidx_v)
    pltpu.sync_copy(dst_hbm.at[sid], didx_v)

    pltpu.sync_copy(zz_hbm, agg_sp.at[pl.ds(r0, RPT)])
    plsc.subcore_barrier()

    gh = g_hbm.at[cid]

    def chunk(j, _):
        pltpu.async_copy(gh.at[sidx_v.at[j]], row_v, sem).wait()
        pltpu.sync_copy(row_v, agg_sp.at[didx_v.at[j]], add=True)
        return 0

    lax.fori_loop(0, NCHUNK, chunk, 0)
    plsc.subcore_barrier()

    pltpu.sync_copy(agg_sp.at[pl.ds(r0, RPT)], agg_v)
    pltpu.sync_copy(sc_hbm.at[cid, pl.ds(r0, RPT)], scf_v)
    pltpu.sync_copy(b_hbm.at[cid, pl.ds(r0, RPT)], bf_v)

    def upd_row(r, _):
        for k in range(HP // LANES):
            sl = pl.ds(k * LANES, LANES)
            agg_v[r, sl] = scf_v[r, sl] * agg_v[r, sl] + bf_v[r, sl]
        return 0

    lax.fori_loop(0, RPT, upd_row, 0)
    pltpu.sync_copy(agg_v, out_hbm.at[cid, pl.ds(r0, RPT)])


def _prop_call(g, scale, bias, src3, dst3, zz):
    return pl.kernel(
        _prop_body,
        out_type=jax.ShapeDtypeStruct((NC, NP, HP), jnp.float32),
        mesh=_mesh(),
        scratch_types=[
            pltpu.VMEM((NCHUNK, CH), jnp.int32),
            pltpu.VMEM((NCHUNK, CH), jnp.int32),
            pltpu.VMEM((CH, HP), jnp.float32),
            pltpu.VMEM((RPT, HP), jnp.float32),
            pltpu.VMEM((RPT, HP), jnp.float32),
            pltpu.VMEM((RPT, HP), jnp.float32),
            pltpu.VMEM_SHARED((NP, HP), jnp.float32),
            pltpu.SemaphoreType.DMA,
        ],
        compiler_params=pltpu.CompilerParams(needs_layout_passes=False,
                                             use_tc_tiling_on_sc=False),
    )(g, scale, bias, src3, dst3, zz)


def _padhalves(a):
    pad = ((0, 0), (0, HP - HALF))
    return jnp.stack([jnp.pad(a[:, :HALF], pad),
                      jnp.pad(a[:, HALF:], pad)], axis=0)


def kernel(features, edge_index, W1, b1, W2, b2):
    with _enable_x64(False):
        res = _kernel_impl(features, edge_index, W1, b1, W2, b2)
    return res.astype(jnp.float64)


def _kernel_impl(features, edge_index, W1, b1, W2, b2):
    src = edge_index[0].astype(jnp.int32)
    dst = edge_index[1].astype(jnp.int32)
    pad = EP - E
    fill = jnp.full((pad,), NP - 1, jnp.int32)
    src3 = jnp.concatenate([src, fill]).reshape(NS, NCHUNK, CH)
    dst3 = jnp.concatenate([dst, fill]).reshape(NS, NCHUNK, CH)
    xp = jnp.pad(features.astype(jnp.float32), ((0, NP - N), (0, 0)))
    zz = jnp.zeros((RPT, HP), jnp.float32)

    cnt = _count_call(src3, dst3)
    cntT = jnp.transpose(cnt, (2, 1, 0))
    h0, g0, scarr, fsarr = _mlp_call(
        xp, W1.astype(jnp.float32), b1.astype(jnp.float32).reshape(1, -1),
        W2.astype(jnp.float32), b2.astype(jnp.float32).reshape(1, -1), cntT)

    g = _padhalves(g0)
    sch = _padhalves(scarr)
    biash = ALPHA * g
    fsh = _padhalves(fsarr)
    fbh = ALPHA * _padhalves(h0)

    for _ in range(K - 1):
        g = _prop_call(g, sch, biash, src3, dst3, zz)
    out = _prop_call(g, fsh, fbh, src3, dst3, zz)
    return jnp.concatenate([out[0, :N, :HALF], out[1, :N, :HALF]], axis=1)


# speedup vs baseline: 255.9270x; 1.1708x over previous
import jax
import jax.numpy as jnp
from jax import lax
from jax.experimental import pallas as pl
from jax.experimental.pallas import tpu as pltpu
from jax.experimental.pallas import tpu_sc as plsc
from jax._src.config import enable_x64 as _enable_x64

N = 10000
E = 320000
IN_FEATS = 128
HIDDEN = 128
N_CLASSES = 40
ALPHA = 0.1
K = 10

NC = 2
NS = 16
NW = NC * NS
CH = 128
NCHUNK = 158
EPT = CH * NCHUNK
EP = EPT * NS
NP = 10240
RPT = NP // NS
HALF = N_CLASSES // 2
HP = 32
LANES = 16
I32 = jnp.int32


def _mesh():
    return plsc.VectorSubcoreMesh(core_axis_name="c", subcore_axis_name="s")


def _count_body(src_hbm, dst_hbm, out_hbm, sidx_v, didx_v, cs_v, cd_v):
    cid = lax.axis_index("c")
    sid = lax.axis_index("s")
    wid = cid * I32(NS) + sid

    pltpu.sync_copy(src_hbm.at[sid], sidx_v)
    pltpu.sync_copy(dst_hbm.at[sid], didx_v)

    zeros = jnp.zeros((LANES,), jnp.float32)
    ones = jnp.ones((LANES,), jnp.float32)

    def zero_row(i, _):
        cs_v[pl.ds(i * I32(LANES), LANES)] = zeros
        cd_v[pl.ds(i * I32(LANES), LANES)] = zeros
        return 0

    lax.fori_loop(0, NP // LANES, zero_row, 0)

    def count_chunk(j, _):
        def one_vec(k, _):
            s = sidx_v[j, pl.ds(k * I32(LANES), LANES)]
            d = didx_v[j, pl.ds(k * I32(LANES), LANES)]
            plsc.addupdate_scatter(cs_v, [s], ones)
            plsc.addupdate_scatter(cd_v, [d], ones)
            return 0
        return lax.fori_loop(0, CH // LANES, one_vec, 0)

    lax.fori_loop(0, NCHUNK, count_chunk, 0)
    pltpu.sync_copy(cs_v, out_hbm.at[wid, 0])
    pltpu.sync_copy(cd_v, out_hbm.at[wid, 1])


def _count_call(src3, dst3):
    return pl.kernel(
        _count_body,
        out_type=jax.ShapeDtypeStruct((NW, 2, NP), jnp.float32),
        mesh=_mesh(),
        scratch_types=[
            pltpu.VMEM((NCHUNK, CH), jnp.int32),
            pltpu.VMEM((NCHUNK, CH), jnp.int32),
            pltpu.VMEM((NP,), jnp.float32),
            pltpu.VMEM((NP,), jnp.float32),
        ],
        compiler_params=pltpu.CompilerParams(needs_layout_passes=False),
    )(src3, dst3)


def _mlp_body(x_ref, w1_ref, b1_ref, w2_ref, b2_ref, cnt_ref,
              h0_ref, g0_ref, sc_ref, fs_ref):
    x = x_ref[...]
    h = jnp.maximum(jnp.dot(x, w1_ref[...],
                            preferred_element_type=jnp.float32)
                    + b1_ref[...], 0.0)
    h0 = jnp.dot(h, w2_ref[...], preferred_element_type=jnp.float32) \
        + b2_ref[...]
    tot = 0.5 * jnp.sum(cnt_ref[...], axis=2)
    tot = jnp.maximum(tot, 1.0)
    out_norm = lax.rsqrt(tot[:, 0:1])
    in_norm = lax.rsqrt(tot[:, 1:2])
    h0_ref[...] = h0
    g0_ref[...] = h0 * out_norm
    sc_ref[...] = jnp.broadcast_to((1.0 - ALPHA) * out_norm * in_norm,
                                   h0.shape)
    fs_ref[...] = jnp.broadcast_to((1.0 - ALPHA) * in_norm, h0.shape)


def _mlp_call(xp, W1, b1, W2, b2, cntT):
    BLK = 512
    grid = NP // BLK
    ospec = pl.BlockSpec((BLK, N_CLASSES), lambda i: (i, 0))
    oshape = jax.ShapeDtypeStruct((NP, N_CLASSES), jnp.float32)
    return pl.pallas_call(
        _mlp_body,
        grid=(grid,),
        in_specs=[
            pl.BlockSpec((BLK, IN_FEATS), lambda i: (i, 0)),
            pl.BlockSpec((IN_FEATS, HIDDEN), lambda i: (0, 0)),
            pl.BlockSpec((1, HIDDEN), lambda i: (0, 0)),
            pl.BlockSpec((HIDDEN, N_CLASSES), lambda i: (0, 0)),
            pl.BlockSpec((1, N_CLASSES), lambda i: (0, 0)),
            pl.BlockSpec((BLK, 2, NW), lambda i: (i, 0, 0)),
        ],
        out_specs=[ospec, ospec, ospec, ospec],
        out_shape=[oshape, oshape, oshape, oshape],
    )(xp, W1, b1, W2, b2, cntT)


def _prop_body(g_hbm, sc_hbm, b_hbm, src_hbm, dst_hbm, zz_hbm, out_hbm,
               sidx_v, didx_v, row_v, row_w, agg_v, scf_v, bf_v, agg_sp,
               sem, sem2):
    cid = lax.axis_index("c")
    sid = lax.axis_index("s")
    wid = cid * I32(NS) + sid
    r0 = sid * I32(RPT)

    pltpu.sync_copy(src_hbm.at[sid], sidx_v)
    pltpu.sync_copy(dst_hbm.at[sid], didx_v)

    pltpu.sync_copy(zz_hbm, agg_sp.at[pl.ds(r0, RPT)])
    plsc.subcore_barrier()

    gh = g_hbm.at[cid]

    pltpu.async_copy(gh.at[sidx_v.at[0]], row_v, sem)

    def chunk2(j2, _):
        a = j2 * I32(2)
        b = a + I32(1)
        pltpu.make_async_copy(gh.at[sidx_v.at[a]], row_v, sem).wait()
        pltpu.async_copy(gh.at[sidx_v.at[b]], row_w, sem2)
        pltpu.sync_copy(row_v, agg_sp.at[didx_v.at[a]], add=True)
        pltpu.make_async_copy(gh.at[sidx_v.at[b]], row_w, sem2).wait()

        @pl.when(b + I32(1) < NCHUNK)
        def _():
            pltpu.async_copy(gh.at[sidx_v.at[b + I32(1)]], row_v, sem)

        pltpu.sync_copy(row_w, agg_sp.at[didx_v.at[b]], add=True)
        return 0

    lax.fori_loop(0, NCHUNK // 2, chunk2, 0)
    plsc.subcore_barrier()

    pltpu.sync_copy(agg_sp.at[pl.ds(r0, RPT)], agg_v)
    pltpu.sync_copy(sc_hbm.at[cid, pl.ds(r0, RPT)], scf_v)
    pltpu.sync_copy(b_hbm.at[cid, pl.ds(r0, RPT)], bf_v)

    def upd_row(r, _):
        for k in range(HP // LANES):
            sl = pl.ds(k * LANES, LANES)
            agg_v[r, sl] = scf_v[r, sl] * agg_v[r, sl] + bf_v[r, sl]
        return 0

    lax.fori_loop(0, RPT, upd_row, 0)
    pltpu.sync_copy(agg_v, out_hbm.at[cid, pl.ds(r0, RPT)])


def _prop_call(g, scale, bias, src3, dst3, zz):
    return pl.kernel(
        _prop_body,
        out_type=jax.ShapeDtypeStruct((NC, NP, HP), jnp.float32),
        mesh=_mesh(),
        scratch_types=[
            pltpu.VMEM((NCHUNK, CH), jnp.int32),
            pltpu.VMEM((NCHUNK, CH), jnp.int32),
            pltpu.VMEM((CH, HP), jnp.float32),
            pltpu.VMEM((CH, HP), jnp.float32),
            pltpu.VMEM((RPT, HP), jnp.float32),
            pltpu.VMEM((RPT, HP), jnp.float32),
            pltpu.VMEM((RPT, HP), jnp.float32),
            pltpu.VMEM_SHARED((NP, HP), jnp.float32),
            pltpu.SemaphoreType.DMA,
            pltpu.SemaphoreType.DMA,
        ],
        compiler_params=pltpu.CompilerParams(needs_layout_passes=False,
                                             use_tc_tiling_on_sc=False),
    )(g, scale, bias, src3, dst3, zz)


def _padhalves(a):
    pad = ((0, 0), (0, HP - HALF))
    return jnp.stack([jnp.pad(a[:, :HALF], pad),
                      jnp.pad(a[:, HALF:], pad)], axis=0)


def kernel(features, edge_index, W1, b1, W2, b2):
    with _enable_x64(False):
        res = _kernel_impl(features, edge_index, W1, b1, W2, b2)
    return res.astype(jnp.float64)


def _kernel_impl(features, edge_index, W1, b1, W2, b2):
    src = edge_index[0].astype(jnp.int32)
    dst = edge_index[1].astype(jnp.int32)
    pad = EP - E
    fill = jnp.full((pad,), NP - 1, jnp.int32)
    src3 = jnp.concatenate([src, fill]).reshape(NS, NCHUNK, CH)
    dst3 = jnp.concatenate([dst, fill]).reshape(NS, NCHUNK, CH)
    xp = jnp.pad(features.astype(jnp.float32), ((0, NP - N), (0, 0)))
    zz = jnp.zeros((RPT, HP), jnp.float32)

    cnt = _count_call(src3, dst3)
    cntT = jnp.transpose(cnt, (2, 1, 0))
    h0, g0, scarr, fsarr = _mlp_call(
        xp, W1.astype(jnp.float32), b1.astype(jnp.float32).reshape(1, -1),
        W2.astype(jnp.float32), b2.astype(jnp.float32).reshape(1, -1), cntT)

    g = _padhalves(g0)
    sch = _padhalves(scarr)
    biash = ALPHA * g
    fsh = _padhalves(fsarr)
    fbh = ALPHA * _padhalves(h0)

    for _ in range(K - 1):
        g = _prop_call(g, sch, biash, src3, dst3, zz)
    out = _prop_call(g, fsh, fbh, src3, dst3, zz)
    return jnp.concatenate([out[0, :N, :HALF], out[1, :N, :HALF]], axis=1)
